# baseline (speedup 1.0000x reference)
import functools

import jax
import jax.numpy as jnp
from jax import lax
from jax.experimental import pallas as pl
from jax.experimental.pallas import tpu as pltpu
from jax.experimental.pallas import tpu_sc as plsc

_INFO = plsc.get_sparse_core_info()
_NC = _INFO.num_cores
_NS = _INFO.num_subcores
_NW = _NC * _NS


def _make_sc_copy_concat(B, D, R):
    assert B % _NW == 0
    rpw = B // _NW
    assert rpw % R == 0
    n_chunks = rpw // R
    mesh = plsc.VectorSubcoreMesh(core_axis_name="c", subcore_axis_name="s")

    @functools.partial(
        pl.kernel,
        mesh=mesh,
        out_type=jax.ShapeDtypeStruct((B, 2 * D), jnp.float32),
        scratch_types=[
            pltpu.VMEM((R,), jnp.int32),
            pltpu.VMEM((R, D), jnp.float32),
            pltpu.VMEM((R, D), jnp.float32),
            pltpu.SemaphoreType.DMA,
        ],
    )
    def k(x_hbm, pos_hbm, idx_hbm, out_hbm, idx_v, pos_v, x_v, sem):
        wid = lax.axis_index("s") * _NC + lax.axis_index("c")
        base = wid * rpw
        pltpu.sync_copy(idx_hbm, idx_v)
        pltpu.async_copy(pos_hbm.at[idx_v], pos_v, sem).wait()
        for c in range(n_chunks):
            row0 = base + c * R
            pltpu.sync_copy(x_hbm.at[pl.ds(row0, R), :], x_v)
            pltpu.sync_copy(x_v, out_hbm.at[pl.ds(row0, R), pl.ds(0, D)])
            pltpu.sync_copy(pos_v, out_hbm.at[pl.ds(row0, R), pl.ds(D, D)])

    return k


def kernel(x, pos_table, index):
    B, _, D = x.shape
    x2 = x.reshape(B, D)
    pos2 = pos_table.reshape(pos_table.shape[1], D)
    R = 128
    idx = jnp.broadcast_to(jnp.asarray(index, jnp.int32).reshape(1), (R,))
    out = _make_sc_copy_concat(B, D, R)(x2, pos2, idx)
    return out.reshape(B, 1, 2 * D)
